# baseline (speedup 1.0000x reference)
import functools

import numpy as np
import jax
import jax.numpy as jnp
from jax import lax
from jax.experimental import pallas as pl
from jax.experimental.pallas import tpu as pltpu
from jax.experimental.pallas import tpu_sc as plsc

_B, _S, _D, _H, _DH = 2, 8192, 768, 12, 64
_E = 500
_EP = 512
_NW = 32
_GPW = (_B * 2 * _EP) // _NW
_RPW = (_B * _S) // _NW
_K = 32
_F_TC = 8192
_SCALE = _DH ** -0.5


def _halton(b, n):
    h, d = 0, 1
    seq = []
    for _ in range(n):
        x = d - h
        if x == 1:
            h = 1
            d *= b
        else:
            y = d // b
            while x <= y:
                y //= b
            h = (b + 1) * y - x
        seq.append(h / d)
    return np.array(seq, dtype=np.float64)


def _build_constants():
    n = min(500, int(0.01 * _S * _S))
    h2 = _halton(2, n)
    h3 = _halton(3, n)
    src = (h2 * _S).astype(np.int64)
    dst = (h3 * _S).astype(np.int64)
    keep = src != dst
    src = src[keep][:n]
    dst = dst[keep][:n]
    assert src.shape[0] == _E
    assert np.unique(src).size == _E

    gidx = np.zeros((2, _B, _EP), dtype=np.int32)
    for b in range(_B):
        gidx[0, b, :_E] = b * _S + src
        gidx[0, b, _E:] = b * _S
        gidx[1, b, :_E] = b * _S + dst
        gidx[1, b, _E:] = b * _S
    gidx = gidx.reshape(-1)

    didx_all = np.concatenate(
        [b * _EP + np.arange(_E, dtype=np.int32) for b in range(_B)])
    sidx_all = np.concatenate(
        [(b * _S + src).astype(np.int32) for b in range(_B)])
    didx_w = np.zeros((_NW, _K), dtype=np.int32)
    sidx_w = np.zeros((_NW, _K), dtype=np.int32)
    for w in range(_NW):
        lo, hi = w * _RPW, (w + 1) * _RPW
        m = (sidx_all >= lo) & (sidx_all < hi)
        d, s = didx_all[m], sidx_all[m]
        assert d.size <= _K
        free = np.setdiff1d(np.arange(lo, hi, dtype=np.int32), s)[0]
        didx_w[w, :d.size] = d
        sidx_w[w, :s.size] = s
        didx_w[w, d.size:] = _E
        sidx_w[w, s.size:] = free
    return gidx, didx_w.reshape(-1), sidx_w.reshape(-1)


_GIDX_NP, _DIDX_NP, _SIDX_NP = _build_constants()

def _sc_gather_body(x_hbm, idx_hbm, out_hbm, idx_v, rows_v, sem):
    wid = lax.axis_index("s") * 2 + lax.axis_index("c")
    base = wid * _GPW
    pltpu.sync_copy(idx_hbm.at[pl.ds(base, _GPW)], idx_v)
    pltpu.async_copy(x_hbm.at[idx_v], rows_v, sem).wait()
    pltpu.sync_copy(rows_v, out_hbm.at[pl.ds(base, _GPW)])


def _sc_fill_body(tile_hbm, out_ref, fill_v, fsem):
    wid = lax.axis_index("s") * 2 + lax.axis_index("c")
    rows = (_B * _S - _F_TC) // _NW
    base = _F_TC + wid * rows
    pltpu.sync_copy(tile_hbm, fill_v)
    fills = [
        pltpu.async_copy(fill_v, out_ref.at[pl.ds(base + j * 64, 64)], fsem)
        for j in range(rows // 64)
    ]
    for c in fills:
        c.wait()


def _sc_scatter_body(didx_hbm, sidx_hbm, delta_hbm, out_ref,
                     didx_v, sidx_v, rows_v, sem):
    wid = lax.axis_index("s") * 2 + lax.axis_index("c")
    pltpu.sync_copy(didx_hbm.at[pl.ds(wid * _K, _K)], didx_v)
    pltpu.sync_copy(sidx_hbm.at[pl.ds(wid * _K, _K)], sidx_v)
    pltpu.async_copy(delta_hbm.at[didx_v], rows_v, sem).wait()
    pltpu.async_copy(rows_v, out_ref.at[sidx_v], sem).wait()


@functools.lru_cache(maxsize=None)
def _sc_kernels():
    mesh = plsc.VectorSubcoreMesh(core_axis_name="c", subcore_axis_name="s")
    gather = pl.kernel(
        _sc_gather_body,
        out_type=jax.ShapeDtypeStruct((_B * 2 * _EP, _D), jnp.float32),
        mesh=mesh,
        scratch_types=[
            pltpu.VMEM((_GPW,), jnp.int32),
            pltpu.VMEM((_GPW, _D), jnp.float32),
            pltpu.SemaphoreType.DMA,
        ],
    )
    fill = pl.kernel(
        _sc_fill_body,
        out_type=(),
        mesh=mesh,
        scratch_types=[
            pltpu.VMEM((64, _D), jnp.float32),
            pltpu.SemaphoreType.DMA,
        ],
    )
    scatter = pl.kernel(
        _sc_scatter_body,
        out_type=(),
        mesh=mesh,
        scratch_types=[
            pltpu.VMEM((_K,), jnp.int32),
            pltpu.VMEM((_K,), jnp.int32),
            pltpu.VMEM((_K, _D), jnp.float32),
            pltpu.SemaphoreType.DMA,
        ],
    )
    return gather, fill, scatter


def _tc_fill_body(bo_ref, out_ref):
    out_ref[...] = jnp.broadcast_to(bo_ref[...], (_RPW, _D))


_tc_fill = pl.pallas_call(
    _tc_fill_body,
    grid=(_F_TC // _RPW,),
    in_specs=[pl.BlockSpec((1, _D), lambda i: (0, 0))],
    out_specs=pl.BlockSpec((_RPW, _D), lambda i: (i, 0)),
    out_shape=jax.ShapeDtypeStruct((_B * _S, _D), jnp.float32),
)


def _mktile_body(bo_ref, tile_ref):
    tile_ref[...] = jnp.broadcast_to(bo_ref[...], (64, _D))


_mktile = pl.pallas_call(
    _mktile_body,
    out_shape=jax.ShapeDtypeStruct((64, _D), jnp.float32),
)


def _tc_body(xg_ref, wq_ref, wk_ref, wv_ref, wo_ref, bq_ref, bk_ref, bv_ref,
             bo_ref, w1_ref, b1_ref, w2_ref, b2_ref, ew_ref, delta_ref):
    dn = (((1,), (1,)), ((), ()))
    hi = lax.Precision.DEFAULT
    M = _B * _EP
    xs = xg_ref[:M, :]
    xd = xg_ref[M:, :]
    qs = lax.dot_general(xs, wq_ref[...], dn, precision=hi,
                         preferred_element_type=jnp.float32) + bq_ref[...]
    kd = lax.dot_general(xd, wk_ref[...], dn, precision=hi,
                         preferred_element_type=jnp.float32) + bk_ref[...]
    vd = lax.dot_general(xd, wv_ref[...], dn, precision=hi,
                         preferred_element_type=jnp.float32) + bv_ref[...]
    w1 = w1_ref[...]
    w1a = w1[:, :_DH]
    w1b = w1[:, _DH:]
    b1 = b1_ref[...]
    w2 = w2_ref[...]
    b2 = b2_ref[0, 0]
    es_cols = []
    for h in range(_H):
        sl = slice(h * _DH, (h + 1) * _DH)
        pre = (lax.dot_general(qs[:, sl], w1a, dn, precision=hi,
                               preferred_element_type=jnp.float32)
               + lax.dot_general(kd[:, sl], w1b, dn, precision=hi,
                                 preferred_element_type=jnp.float32) + b1)
        hm = 0.5 * pre * (1.0 + lax.erf(pre * (2.0 ** -0.5)))
        es_cols.append(jnp.sum(hm * w2, axis=1, keepdims=True) + b2)
    es = jnp.concatenate(es_cols, axis=1) * _SCALE
    valid = lax.rem(lax.broadcasted_iota(jnp.int32, (M, 1), 0), _EP) < _E
    es = jnp.where(valid, es, -1e30)
    ps = []
    for b in range(_B):
        esb = es[b * _EP:(b + 1) * _EP, :]
        mb = jnp.max(esb, axis=0, keepdims=True)
        ps.append(jnp.exp(esb - mb))
    p = jnp.where(valid, jnp.concatenate(ps, axis=0), 0.0)
    den_cols = []
    for b in range(_B):
        pb = p[b * _EP:(b + 1) * _EP, :]
        den_cols.append(jnp.broadcast_to(
            jnp.sum(pb, axis=0, keepdims=True), (_EP, _H)))
    ea = (p / jnp.concatenate(den_cols, axis=0)) * ew_ref[...]
    row = jnp.concatenate(
        [ea[:, h:h + 1] * vd[:, h * _DH:(h + 1) * _DH] for h in range(_H)],
        axis=1)
    delta_ref[...] = lax.dot_general(row, wo_ref[...], dn, precision=hi,
                                     preferred_element_type=jnp.float32) \
        + bo_ref[...]


_full = lambda shape: pl.BlockSpec(shape, lambda: (0,) * len(shape))

_tc_compute = pl.pallas_call(
    _tc_body,
    in_specs=[
        _full((2 * _B * _EP, _D)),
        _full((_D, _D)), _full((_D, _D)), _full((_D, _D)), _full((_D, _D)),
        _full((1, _D)), _full((1, _D)), _full((1, _D)), _full((1, _D)),
        _full((_DH, 2 * _DH)), _full((1, _DH)), _full((1, _DH)),
        _full((1, 1)), _full((1, _H)),
    ],
    out_specs=_full((_B * _EP, _D)),
    out_shape=jax.ShapeDtypeStruct((_B * _EP, _D), jnp.float32),
)


def kernel(x, Wq, bq, Wk, bk, Wv, bv, Wo, bo, edge_weight, W1, b1, W2, b2):
    B, S, D = x.shape
    assert (B, S, D) == (_B, _S, _D)
    _sc_gather, _sc_fill, _sc_scatter = _sc_kernels()
    xg = _sc_gather(x.reshape(B * S, D), jnp.asarray(_GIDX_NP))
    filled = _tc_fill(bo.reshape(1, D))
    out_ref = jax.new_ref(filled)
    _sc_fill(_mktile(bo.reshape(1, D)), out_ref)
    delta = _tc_compute(
        xg, Wq, Wk, Wv, Wo,
        bq.reshape(1, D), bk.reshape(1, D), bv.reshape(1, D),
        bo.reshape(1, D), W1, b1.reshape(1, _DH), W2.reshape(1, _DH),
        b2.reshape(1, 1), edge_weight.reshape(1, _H))
    _sc_scatter(jnp.asarray(_DIDX_NP), jnp.asarray(_SIDX_NP), delta, out_ref)
    return jax.freeze(out_ref).reshape(B, S, D)


# speedup vs baseline: 1.0873x; 1.0873x over previous
import functools

import numpy as np
import jax
import jax.numpy as jnp
from jax import lax
from jax.experimental import pallas as pl
from jax.experimental.pallas import tpu as pltpu
from jax.experimental.pallas import tpu_sc as plsc

_B, _S, _D, _H, _DH = 2, 8192, 768, 12, 64
_E = 500
_EP = 512
_NW = 32
_GPW = (_B * 2 * _EP) // _NW
_RPW = (_B * _S) // _NW
_K = 32
_F_TC = 4096
_SCALE = _DH ** -0.5


def _halton(b, n):
    h, d = 0, 1
    seq = []
    for _ in range(n):
        x = d - h
        if x == 1:
            h = 1
            d *= b
        else:
            y = d // b
            while x <= y:
                y //= b
            h = (b + 1) * y - x
        seq.append(h / d)
    return np.array(seq, dtype=np.float64)


def _build_constants():
    n = min(500, int(0.01 * _S * _S))
    h2 = _halton(2, n)
    h3 = _halton(3, n)
    src = (h2 * _S).astype(np.int64)
    dst = (h3 * _S).astype(np.int64)
    keep = src != dst
    src = src[keep][:n]
    dst = dst[keep][:n]
    assert src.shape[0] == _E
    assert np.unique(src).size == _E

    gidx = np.zeros((2, _B, _EP), dtype=np.int32)
    for b in range(_B):
        gidx[0, b, :_E] = b * _S + src
        gidx[0, b, _E:] = b * _S
        gidx[1, b, :_E] = b * _S + dst
        gidx[1, b, _E:] = b * _S
    gidx = gidx.reshape(-1)

    didx_all = np.concatenate(
        [b * _EP + np.arange(_E, dtype=np.int32) for b in range(_B)])
    sidx_all = np.concatenate(
        [(b * _S + src).astype(np.int32) for b in range(_B)])
    didx_w = np.zeros((_NW, _K), dtype=np.int32)
    sidx_w = np.zeros((_NW, _K), dtype=np.int32)
    for w in range(_NW):
        lo, hi = w * _RPW, (w + 1) * _RPW
        m = (sidx_all >= lo) & (sidx_all < hi)
        d, s = didx_all[m], sidx_all[m]
        assert d.size <= _K
        free = np.setdiff1d(np.arange(lo, hi, dtype=np.int32), s)[0]
        didx_w[w, :d.size] = d
        sidx_w[w, :s.size] = s
        didx_w[w, d.size:] = _E
        sidx_w[w, s.size:] = free
    return gidx, didx_w.reshape(-1), sidx_w.reshape(-1)


_GIDX_NP, _DIDX_NP, _SIDX_NP = _build_constants()

def _sc_gather_body(x_hbm, idx_hbm, out_hbm, idx_v, rows_v, sem):
    wid = lax.axis_index("s") * 2 + lax.axis_index("c")
    base = wid * _GPW
    pltpu.sync_copy(idx_hbm.at[pl.ds(base, _GPW)], idx_v)
    pltpu.async_copy(x_hbm.at[idx_v], rows_v, sem).wait()
    pltpu.sync_copy(rows_v, out_hbm.at[pl.ds(base, _GPW)])


def _sc_fill_body(tile_hbm, out_ref, fill_v, fsem):
    wid = lax.axis_index("s") * 2 + lax.axis_index("c")
    rows = (_B * _S - _F_TC) // _NW
    base = _F_TC + wid * rows
    pltpu.sync_copy(tile_hbm, fill_v)
    fills = [
        pltpu.async_copy(fill_v, out_ref.at[pl.ds(base + j * 64, 64)], fsem)
        for j in range(rows // 64)
    ]
    for c in fills:
        c.wait()


def _sc_scatter_body(didx_hbm, sidx_hbm, delta_hbm, out_ref,
                     didx_v, sidx_v, rows_v, sem):
    wid = lax.axis_index("s") * 2 + lax.axis_index("c")
    pltpu.sync_copy(didx_hbm.at[pl.ds(wid * _K, _K)], didx_v)
    pltpu.sync_copy(sidx_hbm.at[pl.ds(wid * _K, _K)], sidx_v)
    pltpu.async_copy(delta_hbm.at[didx_v], rows_v, sem).wait()
    pltpu.async_copy(rows_v, out_ref.at[sidx_v], sem).wait()


@functools.lru_cache(maxsize=None)
def _sc_kernels():
    mesh = plsc.VectorSubcoreMesh(core_axis_name="c", subcore_axis_name="s")
    gather = pl.kernel(
        _sc_gather_body,
        out_type=jax.ShapeDtypeStruct((_B * 2 * _EP, _D), jnp.float32),
        mesh=mesh,
        scratch_types=[
            pltpu.VMEM((_GPW,), jnp.int32),
            pltpu.VMEM((_GPW, _D), jnp.float32),
            pltpu.SemaphoreType.DMA,
        ],
    )
    fill = pl.kernel(
        _sc_fill_body,
        out_type=(),
        mesh=mesh,
        scratch_types=[
            pltpu.VMEM((64, _D), jnp.float32),
            pltpu.SemaphoreType.DMA,
        ],
    )
    scatter = pl.kernel(
        _sc_scatter_body,
        out_type=(),
        mesh=mesh,
        scratch_types=[
            pltpu.VMEM((_K,), jnp.int32),
            pltpu.VMEM((_K,), jnp.int32),
            pltpu.VMEM((_K, _D), jnp.float32),
            pltpu.SemaphoreType.DMA,
        ],
    )
    return gather, fill, scatter


def _tc_fill_body(bo_ref, out_ref, tile_ref):
    out_ref[...] = jnp.broadcast_to(bo_ref[...], (_RPW, _D))
    tile_ref[...] = jnp.broadcast_to(bo_ref[...], (64, _D))


_tc_fill = pl.pallas_call(
    _tc_fill_body,
    grid=(_F_TC // _RPW,),
    in_specs=[pl.BlockSpec((1, _D), lambda i: (0, 0))],
    out_specs=[
        pl.BlockSpec((_RPW, _D), lambda i: (i, 0)),
        pl.BlockSpec((64, _D), lambda i: (0, 0)),
    ],
    out_shape=[
        jax.ShapeDtypeStruct((_B * _S, _D), jnp.float32),
        jax.ShapeDtypeStruct((64, _D), jnp.float32),
    ],
)


def _tc_body(xg_ref, wq_ref, wk_ref, wv_ref, wo_ref, bq_ref, bk_ref, bv_ref,
             bo_ref, w1_ref, b1_ref, w2_ref, b2_ref, ew_ref, delta_ref):
    dn = (((1,), (1,)), ((), ()))
    hi = lax.Precision.DEFAULT
    M = _B * _EP
    xs = xg_ref[:M, :]
    xd = xg_ref[M:, :]
    qs = lax.dot_general(xs, wq_ref[...], dn, precision=hi,
                         preferred_element_type=jnp.float32) + bq_ref[...]
    kd = lax.dot_general(xd, wk_ref[...], dn, precision=hi,
                         preferred_element_type=jnp.float32) + bk_ref[...]
    vd = lax.dot_general(xd, wv_ref[...], dn, precision=hi,
                         preferred_element_type=jnp.float32) + bv_ref[...]
    w1 = w1_ref[...]
    w1a = w1[:, :_DH]
    w1b = w1[:, _DH:]
    b1 = b1_ref[...]
    w2 = w2_ref[...]
    b2 = b2_ref[0, 0]
    es_cols = []
    for h in range(_H):
        sl = slice(h * _DH, (h + 1) * _DH)
        pre = (lax.dot_general(qs[:, sl], w1a, dn, precision=hi,
                               preferred_element_type=jnp.float32)
               + lax.dot_general(kd[:, sl], w1b, dn, precision=hi,
                                 preferred_element_type=jnp.float32) + b1)
        hm = 0.5 * pre * (1.0 + lax.erf(pre * (2.0 ** -0.5)))
        es_cols.append(jnp.sum(hm * w2, axis=1, keepdims=True) + b2)
    es = jnp.concatenate(es_cols, axis=1) * _SCALE
    valid = lax.rem(lax.broadcasted_iota(jnp.int32, (M, 1), 0), _EP) < _E
    es = jnp.where(valid, es, -1e30)
    ps = []
    for b in range(_B):
        esb = es[b * _EP:(b + 1) * _EP, :]
        mb = jnp.max(esb, axis=0, keepdims=True)
        ps.append(jnp.exp(esb - mb))
    p = jnp.where(valid, jnp.concatenate(ps, axis=0), 0.0)
    den_cols = []
    for b in range(_B):
        pb = p[b * _EP:(b + 1) * _EP, :]
        den_cols.append(jnp.broadcast_to(
            jnp.sum(pb, axis=0, keepdims=True), (_EP, _H)))
    ea = (p / jnp.concatenate(den_cols, axis=0)) * ew_ref[...]
    row = jnp.concatenate(
        [ea[:, h:h + 1] * vd[:, h * _DH:(h + 1) * _DH] for h in range(_H)],
        axis=1)
    delta_ref[...] = lax.dot_general(row, wo_ref[...], dn, precision=hi,
                                     preferred_element_type=jnp.float32) \
        + bo_ref[...]


_full = lambda shape: pl.BlockSpec(shape, lambda: (0,) * len(shape))

_tc_compute = pl.pallas_call(
    _tc_body,
    in_specs=[
        _full((2 * _B * _EP, _D)),
        _full((_D, _D)), _full((_D, _D)), _full((_D, _D)), _full((_D, _D)),
        _full((1, _D)), _full((1, _D)), _full((1, _D)), _full((1, _D)),
        _full((_DH, 2 * _DH)), _full((1, _DH)), _full((1, _DH)),
        _full((1, 1)), _full((1, _H)),
    ],
    out_specs=_full((_B * _EP, _D)),
    out_shape=jax.ShapeDtypeStruct((_B * _EP, _D), jnp.float32),
)


def kernel(x, Wq, bq, Wk, bk, Wv, bv, Wo, bo, edge_weight, W1, b1, W2, b2):
    B, S, D = x.shape
    assert (B, S, D) == (_B, _S, _D)
    _sc_gather, _sc_fill, _sc_scatter = _sc_kernels()
    xg = _sc_gather(x.reshape(B * S, D), jnp.asarray(_GIDX_NP))
    filled, tile = _tc_fill(bo.reshape(1, D))
    out_ref = jax.new_ref(filled)
    _sc_fill(tile, out_ref)
    delta = _tc_compute(
        xg, Wq, Wk, Wv, Wo,
        bq.reshape(1, D), bk.reshape(1, D), bv.reshape(1, D),
        bo.reshape(1, D), W1, b1.reshape(1, _DH), W2.reshape(1, _DH),
        b2.reshape(1, 1), edge_weight.reshape(1, _H))
    _sc_scatter(jnp.asarray(_DIDX_NP), jnp.asarray(_SIDX_NP), delta, out_ref)
    return jax.freeze(out_ref).reshape(B, S, D)
